# baseline (speedup 1.0000x reference)
import functools

import jax
import jax.numpy as jnp
from jax import lax
from jax.experimental import pallas as pl
from jax.experimental.pallas import tpu as pltpu
from jax.experimental.pallas import tpu_sc as plsc

D = 64
DH = 32
BT = 128
W = 128

_info = plsc.get_sparse_core_info()
_NC, _NS = _info.num_cores, _info.num_subcores


@functools.lru_cache(maxsize=None)
def _make_fused(B: int, H: int, V: int):
    n_w = V // W
    rem = n_w % _NS
    p1_main = n_w - rem
    p1_blocks = p1_main // _NS
    btiles = B // BT
    upt = H * btiles // _NS
    IC = 40
    n_chunks = upt // IC
    mesh = plsc.VectorSubcoreMesh(core_axis_name="c", subcore_axis_name="s")

    @functools.partial(
        pl.kernel,
        mesh=mesh,
        compiler_params=pltpu.CompilerParams(
            use_tc_tiling_on_sc=True, needs_layout_passes=False),
        out_type=jax.ShapeDtypeStruct((H, D, B), jnp.float32),
        scratch_types=[
            pltpu.HBM((_NC * V, DH), jnp.float32),
            pltpu.VMEM((2, DH, W), jnp.float32),
            pltpu.VMEM((W, DH + 1), jnp.float32),
            pltpu.VMEM((2, W, DH), jnp.float32),
            pltpu.VMEM((IC, BT), jnp.int32),
            pltpu.VMEM((2, BT, DH), jnp.float32),
            pltpu.VMEM((DH, BT + 1), jnp.float32),
            pltpu.VMEM((2, DH, BT), jnp.float32),
            pltpu.SemaphoreType.DMA,
            pltpu.SemaphoreType.DMA,
            pltpu.SemaphoreType.DMA,
            pltpu.SemaphoreType.DMA,
            pltpu.SemaphoreType.DMA,
            pltpu.SemaphoreType.DMA,
            pltpu.SemaphoreType.DMA,
            pltpu.SemaphoreType.DMA,
            pltpu.SemaphoreType.DMA,
        ],
    )
    def fused(idx_hbm, tabt_hbm, tailt_hbm, out_hbm,
              t_all, p1i, p1o, p1c, idx_all, g, gt, gtc,
              i0, i1, o0, o1, g0, g1, w0, w1, isem):
        c = lax.axis_index("c")
        s = lax.axis_index("s")
        iota = lax.iota(jnp.int32, 16)
        isems = (i0, i1)
        osems = (o0, o1)
        gsems = (g0, g1)
        wsems = (w0, w1)

        def transpose_tile(src, dst, rows, cols, unroll=8):
            def one(i):
                fi = jnp.full((16,), i, jnp.int32)
                for jg in range(cols // 16):
                    v = src[i, pl.ds(jg * 16, 16)]
                    plsc.store_scatter(dst, [jg * 16 + iota, fi], v)

            def body(k, carry):
                for uu in range(unroll):
                    one(k * unroll + uu)
                return carry
            lax.fori_loop(0, rows // unroll, body, 0)

        def compact(src_ref, dst_ref, rows, width):
            def body(d, carry):
                for jg in range(width // 16):
                    sl = pl.ds(jg * 16, 16)
                    dst_ref[d, sl] = src_ref[d, sl]
                return carry
            lax.fori_loop(0, rows, body, 0)

        def p1_t(blk):
            return s + blk * _NS

        def p1_src(t):
            return tabt_hbm.at[pl.ds(c * DH, DH), pl.ds(t * W, W)]

        def p1_start_in(t, slot):
            pltpu.async_copy(p1_src(t), p1i.at[slot], isems[slot])

        def p1_out_copy(t, slot):
            return pltpu.make_async_copy(
                p1c.at[slot], t_all.at[pl.ds(c * V + t * W, W), :],
                osems[slot])

        def p1_work(t, slot, first):
            pltpu.make_async_copy(p1_src(t), p1i.at[slot], isems[slot]).wait()
            if not first:
                p1_out_copy(t, slot).wait()
            transpose_tile(p1i.at[slot], p1o, DH, W)
            compact(p1o, p1c.at[slot], W, DH)
            p1_out_copy(t, slot).start()

        p1_start_in(p1_t(0), 0)
        p1_start_in(p1_t(1), 1)
        p1_work(p1_t(0), 0, True)
        p1_start_in(p1_t(2), 0)
        p1_work(p1_t(1), 1, True)
        p1_start_in(p1_t(3), 1)

        def p1_loop(k, carry):
            b0 = 2 * k
            p1_work(p1_t(b0), 0, False)

            @pl.when(b0 + 2 < p1_blocks)
            def _():
                p1_start_in(p1_t(b0 + 2), 0)
            p1_work(p1_t(b0 + 1), 1, False)

            @pl.when(b0 + 3 < p1_blocks)
            def _():
                p1_start_in(p1_t(b0 + 3), 1)
            return carry

        lax.fori_loop(1, p1_blocks // 2, p1_loop, 0)

        @pl.when(s < rem)
        def _():
            t = p1_main + s
            pltpu.async_copy(p1_src(t), p1i.at[0], isems[0]).wait()
            p1_out_copy(t, 0).wait()
            transpose_tile(p1i.at[0], p1o, DH, W)
            compact(p1o, p1c.at[0], W, DH)
            p1_out_copy(t, 0).start()

        @pl.when(s == rem)
        def _():
            pltpu.async_copy(
                tailt_hbm.at[pl.ds(c * DH, DH), :], gtc.at[0], isem).wait()
            transpose_tile(gtc.at[0], g.at[0], DH, BT)
            pltpu.async_copy(
                g.at[0], t_all.at[pl.ds(c * V + V - BT, BT), :], isem).wait()

        p1_out_copy(0, 0).wait()
        p1_out_copy(0, 1).wait()

        plsc.subcore_barrier()

        def p2_start_gather(k, slot):
            pltpu.async_copy(t_all.at[idx_all.at[k]], g.at[slot], gsems[slot])

        def p2_dst(u):
            h = u // btiles
            bt = u % btiles
            return out_hbm.at[h, pl.ds(c * DH, DH), pl.ds(bt * BT, BT)]

        def p2_out_copy(u, slot):
            return pltpu.make_async_copy(gtc.at[slot], p2_dst(u), wsems[slot])

        def p2_work(k, u, slot, first):
            pltpu.make_async_copy(
                t_all.at[idx_all.at[k]], g.at[slot], gsems[slot]).wait()
            if not first:
                p2_out_copy(u, slot).wait()
            transpose_tile(g.at[slot], gt, BT, DH)
            compact(gt, gtc.at[slot], DH, BT)
            p2_out_copy(u, slot).start()

        def p2_chunk(ci, carry):
            u0 = s * upt + ci * IC
            pltpu.async_copy(
                idx_hbm.at[pl.ds(u0, IC), :], idx_all, isem).wait()

            def bias_row(k, carry):
                for j in range(BT // 16):
                    sl = (k, pl.ds(j * 16, 16))
                    idx_all[sl] = idx_all[sl] + c * V
                return carry

            lax.fori_loop(0, IC, bias_row, 0)

            p2_start_gather(0, 0)
            p2_start_gather(1, 1)
            p2_work(0, u0, 0, True)
            p2_start_gather(2, 0)
            p2_work(1, u0 + 1, 1, True)
            p2_start_gather(3, 1)

            def p2_loop(k, carry):
                k0 = 2 * k
                p2_work(k0, u0 + k0, 0, False)

                @pl.when(k0 + 2 < IC)
                def _():
                    p2_start_gather(k0 + 2, 0)
                p2_work(k0 + 1, u0 + k0 + 1, 1, False)

                @pl.when(k0 + 3 < IC)
                def _():
                    p2_start_gather(k0 + 3, 1)
                return carry

            lax.fori_loop(1, IC // 2, p2_loop, 0)
            p2_out_copy(u0 + IC - 2, 0).wait()
            p2_out_copy(u0 + IC - 1, 1).wait()
            return carry

        lax.fori_loop(0, n_chunks, p2_chunk, 0)

    return fused


def kernel(inputs, table):
    B, H = inputs.shape
    V, _ = table.shape
    idxt = inputs.T.reshape((B * H) // BT, BT)
    tabt = table.T
    out = _make_fused(B, H, V)(idxt, tabt, tabt[:, V - BT:])
    return out.transpose(2, 0, 1)


# speedup vs baseline: 2.2260x; 2.2260x over previous
import functools

import jax
import jax.numpy as jnp
from jax import lax
from jax.experimental import pallas as pl
from jax.experimental.pallas import tpu as pltpu
from jax.experimental.pallas import tpu_sc as plsc

D = 64
CHUNK = 128

_info = plsc.get_sparse_core_info()
_NC, _NS = _info.num_cores, _info.num_subcores
_NW = _NC * _NS


@functools.lru_cache(maxsize=None)
def _make_gather(B: int):
    assert B % (_NW * CHUNK) == 0
    cpw = B // (_NW * CHUNK)
    assert cpw % 2 == 0
    mesh = plsc.VectorSubcoreMesh(core_axis_name="c", subcore_axis_name="s")

    @functools.partial(
        pl.kernel,
        mesh=mesh,
        compiler_params=pltpu.CompilerParams(use_tc_tiling_on_sc=False),
        out_type=jax.ShapeDtypeStruct((B, D), jnp.float32),
        scratch_types=[
            pltpu.VMEM((cpw, CHUNK), jnp.int32),
            pltpu.VMEM((CHUNK, D), jnp.float32),
            pltpu.VMEM((CHUNK, D), jnp.float32),
            pltpu.SemaphoreType.DMA,
            pltpu.SemaphoreType.DMA,
        ],
    )
    def gather_kernel(idx_hbm, table_hbm, out_hbm, idx_v, buf0, buf1,
                      sem0, sem1):
        wid = lax.axis_index("s") * _NC + lax.axis_index("c")
        chunk0 = wid * cpw
        row0 = chunk0 * CHUNK

        pltpu.sync_copy(idx_hbm.at[pl.ds(chunk0, cpw)], idx_v)

        def start_gather(j, buf, sem):
            pltpu.async_copy(table_hbm.at[idx_v.at[j]], buf, sem)

        def wait_gather(buf, sem):
            pltpu.make_async_copy(table_hbm.at[idx_v.at[0]], buf, sem).wait()

        def write_out(j, buf):
            pltpu.sync_copy(buf, out_hbm.at[pl.ds(row0 + j * CHUNK, CHUNK)])

        start_gather(0, buf0, sem0)

        def body(i, carry):
            j0 = 2 * i
            start_gather(j0 + 1, buf1, sem1)
            wait_gather(buf0, sem0)
            write_out(j0, buf0)

            @pl.when(j0 + 2 < cpw)
            def _():
                start_gather(j0 + 2, buf0, sem0)

            wait_gather(buf1, sem1)
            write_out(j0 + 1, buf1)
            return carry

        lax.fori_loop(0, cpw // 2, body, 0)

    return gather_kernel


def kernel(inputs, table):
    B, H = inputs.shape
    n = B * H
    idx2d = inputs.reshape(n // CHUNK, CHUNK)
    out = _make_gather(n)(idx2d, table)
    return out.reshape(B, H, D)
